# baseline (speedup 1.0000x reference)
import functools
import math

import jax
import jax.numpy as jnp
from jax import lax
from jax.experimental import pallas as pl
from jax.experimental.pallas import tpu as pltpu
from jax.experimental.pallas import tpu_sc as plsc

EMB = 64
SCALE = math.sqrt(EMB)

NC = 2
NS = 16
NW = NC * NS
LANES = 16

CH = 128
NBUF = 2


def _make_gather(B):
    assert B % (NW * CH) == 0
    b_per_w = B // NW
    nchunk = b_per_w // CH
    mesh = plsc.VectorSubcoreMesh(
        core_axis_name="c", subcore_axis_name="s", num_cores=NC, num_subcores=NS
    )

    @functools.partial(
        pl.kernel,
        out_type=jax.ShapeDtypeStruct((B, EMB), jnp.float32),
        mesh=mesh,
        compiler_params=pltpu.CompilerParams(use_tc_tiling_on_sc=False),
        scratch_types=[
            pltpu.VMEM((nchunk, CH), jnp.int32),
            pltpu.VMEM((NBUF, CH, EMB), jnp.float32),
            pltpu.SemaphoreType.DMA((NBUF,)),
        ],
    )
    def gather_kernel(tok_hbm, table_hbm, out_hbm, idx_v, rows_v, gsem):
        wid = lax.axis_index("s") * NC + lax.axis_index("c")
        base = wid * b_per_w
        pltpu.sync_copy(tok_hbm.at[wid], idx_v)

        def start_gather(j, b):
            pltpu.async_copy(table_hbm.at[idx_v.at[j]], rows_v.at[b], gsem.at[b])

        def wait_gather(j, b):
            pltpu.make_async_copy(
                table_hbm.at[idx_v.at[j]], rows_v.at[b], gsem.at[b]
            ).wait()

        for b in range(NBUF):
            start_gather(b, b)

        @pl.loop(0, nchunk, step=NBUF)
        def _group(g):
            for b in range(NBUF):
                j = g + b
                wait_gather(j, b)
                buf = rows_v.at[b]

                @pl.loop(0, CH)
                def _scale(r):
                    for c in range(EMB // LANES):
                        sl = pl.ds(c * LANES, LANES)
                        buf[r, sl] = buf[r, sl] * SCALE

                pltpu.sync_copy(buf, out_hbm.at[pl.ds(base + j * CH, CH)])

                nj = j + NBUF

                @pl.when(nj < nchunk)
                def _prefetch():
                    start_gather(nj, b)

    return gather_kernel


def kernel(tokens, table):
    B = tokens.size
    tok = tokens.astype(jnp.int32).reshape(NW, B // NW // CH, CH)
    out = _make_gather(B)(tok, table)
    return out.reshape(tokens.shape + (EMB,))


# speedup vs baseline: 1.0558x; 1.0558x over previous
import functools
import math

import jax
import jax.numpy as jnp
from jax import lax
from jax.experimental import pallas as pl
from jax.experimental.pallas import tpu as pltpu
from jax.experimental.pallas import tpu_sc as plsc

EMB = 64
SCALE = math.sqrt(EMB)

NC = 2
NS = 16
NW = NC * NS
LANES = 16

CH = 128
NBUF = 4


def _make_gather(B):
    assert B % (NW * CH) == 0
    b_per_w = B // NW
    nchunk = b_per_w // CH
    assert nchunk % NBUF == 0
    mesh = plsc.VectorSubcoreMesh(
        core_axis_name="c", subcore_axis_name="s", num_cores=NC, num_subcores=NS
    )

    @functools.partial(
        pl.kernel,
        out_type=jax.ShapeDtypeStruct((B, EMB), jnp.float32),
        mesh=mesh,
        compiler_params=pltpu.CompilerParams(use_tc_tiling_on_sc=False),
        scratch_types=[
            pltpu.VMEM((nchunk, CH), jnp.int32),
            pltpu.VMEM((NBUF, CH, EMB), jnp.float32),
            pltpu.VMEM((NBUF, CH, EMB), jnp.float32),
            pltpu.SemaphoreType.DMA((NBUF,)),
            pltpu.SemaphoreType.DMA((NBUF,)),
        ],
    )
    def gather_kernel(tok_hbm, table_hbm, out_hbm, idx_v, rows_g, rows_s, gsem, ssem):
        wid = lax.axis_index("s") * NC + lax.axis_index("c")
        base = wid * b_per_w
        pltpu.sync_copy(tok_hbm.at[wid], idx_v)

        def start_gather(j, b):
            pltpu.async_copy(table_hbm.at[idx_v.at[j]], rows_g.at[b], gsem.at[b])

        def wait_gather(j, b):
            pltpu.make_async_copy(
                table_hbm.at[idx_v.at[j]], rows_g.at[b], gsem.at[b]
            ).wait()

        def start_store(j, b):
            pltpu.async_copy(
                rows_s.at[b], out_hbm.at[pl.ds(base + j * CH, CH)], ssem.at[b]
            )

        def wait_store(j, b):
            pltpu.make_async_copy(
                rows_s.at[b], out_hbm.at[pl.ds(base + j * CH, CH)], ssem.at[b]
            ).wait()

        for b in range(NBUF):
            start_gather(b, b)

        @pl.loop(0, nchunk, step=NBUF)
        def _group(g):
            for b in range(NBUF):
                j = g + b
                wait_gather(j, b)

                @pl.when(j >= NBUF)
                def _drain():
                    wait_store(j - NBUF, b)

                src = rows_g.at[b]
                dst = rows_s.at[b]

                @plsc.parallel_loop(0, CH, unroll=4)
                def _scale(r):
                    for c in range(EMB // LANES):
                        sl = pl.ds(c * LANES, LANES)
                        dst[r, sl] = src[r, sl] * SCALE

                nj = j + NBUF

                @pl.when(nj < nchunk)
                def _prefetch():
                    start_gather(nj, b)

                start_store(j, b)

        for b in range(NBUF):
            wait_store(nchunk - NBUF + b, b)

    return gather_kernel


def kernel(tokens, table):
    B = tokens.size
    tok = tokens.astype(jnp.int32).reshape(NW, B // NW // CH, CH)
    out = _make_gather(B)(tok, table)
    return out.reshape(tokens.shape + (EMB,))
